# baseline (speedup 1.0000x reference)
import functools

import jax
import jax.numpy as jnp
from jax import lax
from jax.experimental import pallas as pl
from jax.experimental.pallas import tpu as pltpu
from jax.experimental.pallas import tpu_sc as plsc

NC = 2
NS = 16
NW = NC * NS
CHUNK = 128
K = 1
BLK = 512


def _ceil_to(x, m):
    return -(-x // m) * m


def _deg_body(nch, dst2_hbm, zeros1_hbm, deg_hbm, di_v, ones_v, acc):
    c = lax.axis_index("c")
    s = lax.axis_index("s")
    rows = zeros1_hbm.shape[0] // NS
    rs = pl.ds(s * rows, rows)
    for k in range(CHUNK // 16):
        ones_v[pl.ds(k * 16, 16)] = jnp.ones((16,), jnp.float32)
    pltpu.sync_copy(zeros1_hbm.at[rs], acc.at[rs])
    plsc.subcore_barrier()
    rowbase = (c * NS + s) * nch
    pltpu.sync_copy(dst2_hbm.at[pl.ds(rowbase, nch)], di_v)

    def it(i, carry):
        pltpu.sync_copy(ones_v, acc.at[di_v.at[i]], add=True)
        return carry

    lax.fori_loop(0, nch, it, 0)
    plsc.subcore_barrier()
    pltpu.sync_copy(acc.at[rs], deg_hbm.at[c, rs])


J = 2 * K


def _agg_body(nch, hs_hbm, sd2_hbm, zeros2_hbm, p_hbm, sd_v, acc, *bufs):
    rows_b = bufs[:K]
    isem = bufs[K:K + J]
    gsem = bufs[K + J:K + J + K]
    ssem = bufs[K + J + K:]
    c = lax.axis_index("c")
    s = lax.axis_index("s")
    rows = zeros2_hbm.shape[0] // NS
    rs = pl.ds(s * rows, rows)

    @pl.when(c == 0)
    def _():
        pltpu.sync_copy(hs_hbm.at[rs], acc.at[rs])

    @pl.when(c != 0)
    def _():
        pltpu.sync_copy(zeros2_hbm.at[rs], acc.at[rs])

    plsc.subcore_barrier()
    rowbase = (c * NS + s) * nch

    for v in range(J):
        pltpu.async_copy(sd2_hbm.at[rowbase + v], sd_v.at[v], isem[v])
    for v in range(K):
        pltpu.make_async_copy(sd2_hbm.at[rowbase], sd_v.at[v],
                              isem[v]).wait()
        pltpu.async_copy(hs_hbm.at[sd_v.at[v, 0]], rows_b[v], gsem[v])

    def grp(g, carry):
        for v in range(J):
            i = g * J + v
            k = v % K
            j2 = (v + K) % J
            pltpu.make_async_copy(hs_hbm.at[sd_v.at[v, 0]], rows_b[k],
                                  gsem[k]).wait()
            pltpu.async_copy(rows_b[k], acc.at[sd_v.at[v, 1]], ssem[k],
                             add=True)
            pltpu.make_async_copy(rows_b[k], acc.at[sd_v.at[v, 1]],
                                  ssem[k]).wait()
            rl = rowbase + jnp.minimum(i + J, nch - 1)
            pltpu.async_copy(sd2_hbm.at[rl], sd_v.at[v], isem[v])
            pltpu.make_async_copy(sd2_hbm.at[rowbase], sd_v.at[j2],
                                  isem[j2]).wait()
            pltpu.async_copy(hs_hbm.at[sd_v.at[j2, 0]], rows_b[k], gsem[k])
        return carry

    lax.fori_loop(0, nch // J, grp, 0)
    for k in range(K):
        pltpu.make_async_copy(hs_hbm.at[sd_v.at[0, 0]], rows_b[k],
                              gsem[k]).wait()
    for t in range(K, J):
        pltpu.make_async_copy(sd2_hbm.at[rowbase], sd_v.at[t], isem[t]).wait()
    plsc.subcore_barrier()
    pltpu.sync_copy(acc.at[rs], p_hbm.at[c, rs])


def _dinv(degt_blk):
    return lax.rsqrt(1.0 + degt_blk[:, 0:1] + degt_blk[:, 1:2])


def _tc_first_body(x_ref, w_ref, degt_ref, o_ref):
    dinv = _dinv(degt_ref[...])
    o_ref[...] = dinv * jnp.dot(x_ref[...], w_ref[...],
                                preferred_element_type=jnp.float32)


def _tc_mid_body(p_ref, degt_ref, b_ref, w_ref, o_ref):
    dinv = _dinv(degt_ref[...])
    y = jnp.maximum(dinv * (p_ref[0] + p_ref[1]) + b_ref[...], 0.0)
    o_ref[...] = dinv * jnp.dot(y, w_ref[...],
                                preferred_element_type=jnp.float32)


def _tc_last_body(p_ref, degt_ref, b_ref, o_ref):
    dinv = _dinv(degt_ref[...])
    o_ref[...] = dinv * (p_ref[0] + p_ref[1]) + b_ref[...]


def kernel(x, edge_index, W1, b1, W2, b2, W3, b3):
    N, D = x.shape
    E = edge_index.shape[1]
    NP = _ceil_to(N, NS * 8 * 2)
    EP = _ceil_to(E, NW * CHUNK * J)
    nch = EP // (NW * CHUNK)

    x = x.astype(jnp.float32)
    src = edge_index[0].astype(jnp.int32)
    dst = edge_index[1].astype(jnp.int32)
    pad = EP - E
    if pad:
        cyc = jnp.arange(pad, dtype=jnp.int32) % jnp.int32(NP - N)
        src = jnp.concatenate([src, cyc % jnp.int32(N)])
        dst = jnp.concatenate([dst, jnp.int32(N) + cyc])
    src2 = src.reshape(EP // CHUNK, CHUNK)
    dst2 = dst.reshape(EP // CHUNK, CHUNK)
    sd2 = jnp.stack([src2, dst2], axis=1)
    xp = jnp.pad(x, ((0, NP - N), (0, 0)))
    zeros1 = jnp.zeros((NP,), jnp.float32)
    zeros2 = jnp.zeros((NP, D), jnp.float32)

    mesh = plsc.VectorSubcoreMesh(core_axis_name="c", subcore_axis_name="s")

    deg_k = pl.kernel(
        functools.partial(_deg_body, nch),
        out_type=jax.ShapeDtypeStruct((NC, NP), jnp.float32),
        mesh=mesh,
        scratch_types=[
            pltpu.VMEM((nch, CHUNK), jnp.int32),
            pltpu.VMEM((CHUNK,), jnp.float32),
            pltpu.VMEM_SHARED((NP,), jnp.float32),
        ],
    )
    agg_k = pl.kernel(
        functools.partial(_agg_body, nch),
        out_type=jax.ShapeDtypeStruct((NC, NP, D), jnp.float32),
        mesh=mesh,
        scratch_types=[
            pltpu.VMEM((J, 2, CHUNK), jnp.int32),
            pltpu.VMEM_SHARED((NP, D), jnp.float32),
        ] + [pltpu.VMEM((CHUNK, D), jnp.float32)] * K
          + [pltpu.SemaphoreType.DMA] * (J + 2 * K),
    )

    grid = (NP // BLK,) if NP % BLK == 0 else (-(-NP // BLK),)
    row_spec = pl.BlockSpec((BLK, D), lambda i: (i, 0))
    p_spec = pl.BlockSpec((NC, BLK, D), lambda i: (0, i, 0))
    degt_spec = pl.BlockSpec((BLK, NC), lambda i: (i, 0))
    b_spec = pl.BlockSpec((1, D), lambda i: (0, 0))
    w_spec = pl.BlockSpec((D, D), lambda i: (0, 0))

    tc_first = pl.pallas_call(
        _tc_first_body, grid=grid,
        in_specs=[row_spec, w_spec, degt_spec], out_specs=row_spec,
        out_shape=jax.ShapeDtypeStruct((NP, D), jnp.float32))
    tc_mid = pl.pallas_call(
        _tc_mid_body, grid=grid,
        in_specs=[p_spec, degt_spec, b_spec, w_spec], out_specs=row_spec,
        out_shape=jax.ShapeDtypeStruct((NP, D), jnp.float32))
    tc_last = pl.pallas_call(
        _tc_last_body, grid=grid,
        in_specs=[p_spec, degt_spec, b_spec], out_specs=row_spec,
        out_shape=jax.ShapeDtypeStruct((N, D), jnp.float32))

    deg = deg_k(dst2, zeros1)
    degt = deg.T
    hs = tc_first(xp, W1.astype(jnp.float32), degt)
    p = agg_k(hs, sd2, zeros2)
    hs = tc_mid(p, degt, b1.reshape(1, D), W2.astype(jnp.float32))
    p = agg_k(hs, sd2, zeros2)
    hs = tc_mid(p, degt, b2.reshape(1, D), W3.astype(jnp.float32))
    p = agg_k(hs, sd2, zeros2)
    return tc_last(p, degt, b3.reshape(1, D))


# speedup vs baseline: 1.4562x; 1.4562x over previous
import functools

import jax
import jax.numpy as jnp
from jax import lax
from jax.experimental import pallas as pl
from jax.experimental.pallas import tpu as pltpu
from jax.experimental.pallas import tpu_sc as plsc

NC = 2
NS = 16
NW = NC * NS
CHUNK = 128
K = 4
J = 2 * K
BLK = 512


def _ceil_to(x, m):
    return -(-x // m) * m


def _deg_body(nch, dst2_hbm, zeros1_hbm, deg_hbm, di_v, ones_v, acc):
    c = lax.axis_index("c")
    s = lax.axis_index("s")
    rows = zeros1_hbm.shape[0] // NS
    rs = pl.ds(s * rows, rows)
    for k in range(CHUNK // 16):
        ones_v[pl.ds(k * 16, 16)] = jnp.ones((16,), jnp.float32)
    pltpu.sync_copy(zeros1_hbm.at[rs], acc.at[rs])
    plsc.subcore_barrier()
    rowbase = (c * NS + s) * nch
    pltpu.sync_copy(dst2_hbm.at[pl.ds(rowbase, nch)], di_v)

    def it(i, carry):
        pltpu.sync_copy(ones_v, acc.at[di_v.at[i]], add=True)
        return carry

    lax.fori_loop(0, nch, it, 0)
    plsc.subcore_barrier()
    pltpu.sync_copy(acc.at[rs], deg_hbm.at[c, rs])


def _agg_body(nch, npad, hs_hbm, sd3_hbm, p_hbm, sd_v, acc, *bufs):
    rows_b = bufs[:K]
    isem = bufs[K:K + J]
    gsem = bufs[K + J:K + J + K]
    ssem = bufs[K + J + K:]
    c = lax.axis_index("c")
    s = lax.axis_index("s")
    rows = npad // NS
    rs = pl.ds(s * rows, rows)

    pltpu.sync_copy(hs_hbm.at[pl.ds(c * npad + s * rows, rows)], acc.at[rs])
    plsc.subcore_barrier()
    rowbase = s * nch

    for v in range(J):
        pltpu.async_copy(sd3_hbm.at[c, rowbase + v], sd_v.at[v], isem[v])
    for v in range(K):
        pltpu.make_async_copy(sd3_hbm.at[c, rowbase], sd_v.at[v],
                              isem[v]).wait()
        pltpu.async_copy(hs_hbm.at[sd_v.at[v, 0]], rows_b[v], gsem[v])

    def grp(g, carry):
        for v in range(J):
            i = g * J + v
            k = v % K
            j2 = (v + K) % J
            pltpu.make_async_copy(hs_hbm.at[sd_v.at[v, 0]], rows_b[k],
                                  gsem[k]).wait()
            pltpu.async_copy(rows_b[k], acc.at[sd_v.at[v, 1]], ssem[k],
                             add=True)
            pltpu.make_async_copy(rows_b[k], acc.at[sd_v.at[v, 1]],
                                  ssem[k]).wait()
            rl = rowbase + jnp.minimum(i + J, nch - 1)
            pltpu.async_copy(sd3_hbm.at[c, rl], sd_v.at[v], isem[v])
            pltpu.make_async_copy(sd3_hbm.at[c, rowbase], sd_v.at[j2],
                                  isem[j2]).wait()
            pltpu.async_copy(hs_hbm.at[sd_v.at[j2, 0]], rows_b[k], gsem[k])
        return carry

    lax.fori_loop(0, nch // J, grp, 0)
    for k in range(K):
        pltpu.make_async_copy(hs_hbm.at[sd_v.at[0, 0]], rows_b[k],
                              gsem[k]).wait()
    for t in range(K, J):
        pltpu.make_async_copy(sd3_hbm.at[c, rowbase], sd_v.at[t],
                              isem[t]).wait()
    plsc.subcore_barrier()
    pltpu.sync_copy(acc.at[rs], p_hbm.at[c, rs])


def _dinv(degt_blk):
    return lax.rsqrt(1.0 + degt_blk[:, 0:1] + degt_blk[:, 1:2])


def _split_store(o_ref, val, h):
    o_ref[0] = val[:, :h]
    o_ref[1] = val[:, h:]


def _tc_first_body(x_ref, w_ref, degt_ref, o_ref):
    dinv = _dinv(degt_ref[...])
    hs = dinv * jnp.dot(x_ref[...], w_ref[...],
                        preferred_element_type=jnp.float32)
    _split_store(o_ref, hs, w_ref.shape[1] // 2)


def _tc_mid_body(p_ref, degt_ref, b_ref, w_ref, o_ref):
    dinv = _dinv(degt_ref[...])
    agg = jnp.concatenate([p_ref[0], p_ref[1]], axis=1)
    y = jnp.maximum(dinv * agg + b_ref[...], 0.0)
    hs = dinv * jnp.dot(y, w_ref[...], preferred_element_type=jnp.float32)
    _split_store(o_ref, hs, w_ref.shape[1] // 2)


def _tc_last_body(p_ref, degt_ref, b_ref, o_ref):
    dinv = _dinv(degt_ref[...])
    agg = jnp.concatenate([p_ref[0], p_ref[1]], axis=1)
    o_ref[...] = dinv * agg + b_ref[...]


def kernel(x, edge_index, W1, b1, W2, b2, W3, b3):
    N, D = x.shape
    H = D // 2
    E = edge_index.shape[1]
    NP = _ceil_to(N, NS * 8 * 2)
    EP = _ceil_to(E, NS * CHUNK * J)
    nch = EP // (NS * CHUNK)

    x = x.astype(jnp.float32)
    src = edge_index[0].astype(jnp.int32)
    dst = edge_index[1].astype(jnp.int32)
    pad = EP - E
    if pad:
        cyc = jnp.arange(pad, dtype=jnp.int32) % jnp.int32(NP - N)
        src = jnp.concatenate([src, cyc % jnp.int32(N)])
        dst = jnp.concatenate([dst, jnp.int32(N) + cyc])
    src2 = src.reshape(EP // CHUNK, CHUNK)
    dst2 = dst.reshape(EP // CHUNK, CHUNK)
    sd3 = jnp.stack([jnp.stack([src2, dst2], 1),
                     jnp.stack([src2 + jnp.int32(NP), dst2], 1)])
    xp = jnp.pad(x, ((0, NP - N), (0, 0)))
    zeros1 = jnp.zeros((NP,), jnp.float32)

    mesh = plsc.VectorSubcoreMesh(core_axis_name="c", subcore_axis_name="s")

    deg_k = pl.kernel(
        functools.partial(_deg_body, EP // (NW * CHUNK)),
        out_type=jax.ShapeDtypeStruct((NC, NP), jnp.float32),
        mesh=mesh,
        scratch_types=[
            pltpu.VMEM((EP // (NW * CHUNK), CHUNK), jnp.int32),
            pltpu.VMEM((CHUNK,), jnp.float32),
            pltpu.VMEM_SHARED((NP,), jnp.float32),
        ],
    )
    agg_k = pl.kernel(
        functools.partial(_agg_body, nch, NP),
        out_type=jax.ShapeDtypeStruct((NC, NP, H), jnp.float32),
        mesh=mesh,
        compiler_params=pltpu.CompilerParams(use_tc_tiling_on_sc=False),
        scratch_types=[
            pltpu.VMEM((J, 2, CHUNK), jnp.int32),
            pltpu.VMEM_SHARED((NP, H), jnp.float32),
        ] + [pltpu.VMEM((CHUNK, H), jnp.float32)] * K
          + [pltpu.SemaphoreType.DMA] * (J + 2 * K),
    )

    grid = (NP // BLK,) if NP % BLK == 0 else (-(-NP // BLK),)
    row_spec = pl.BlockSpec((BLK, D), lambda i: (i, 0))
    hs_spec = pl.BlockSpec((NC, BLK, H), lambda i: (0, i, 0))
    degt_spec = pl.BlockSpec((BLK, NC), lambda i: (i, 0))
    b_spec = pl.BlockSpec((1, D), lambda i: (0, 0))
    w_spec = pl.BlockSpec((D, D), lambda i: (0, 0))

    tc_first = pl.pallas_call(
        _tc_first_body, grid=grid,
        in_specs=[row_spec, w_spec, degt_spec], out_specs=hs_spec,
        out_shape=jax.ShapeDtypeStruct((NC, NP, H), jnp.float32))
    tc_mid = pl.pallas_call(
        _tc_mid_body, grid=grid,
        in_specs=[hs_spec, degt_spec, b_spec, w_spec], out_specs=hs_spec,
        out_shape=jax.ShapeDtypeStruct((NC, NP, H), jnp.float32))
    tc_last = pl.pallas_call(
        _tc_last_body, grid=grid,
        in_specs=[hs_spec, degt_spec, b_spec], out_specs=row_spec,
        out_shape=jax.ShapeDtypeStruct((N, D), jnp.float32))

    deg = deg_k(dst2, zeros1)
    degt = deg.T
    hs = tc_first(xp, W1.astype(jnp.float32), degt)
    p = agg_k(hs.reshape(NC * NP, H), sd3)
    hs = tc_mid(p, degt, b1.reshape(1, D), W2.astype(jnp.float32))
    p = agg_k(hs.reshape(NC * NP, H), sd3)
    hs = tc_mid(p, degt, b2.reshape(1, D), W3.astype(jnp.float32))
    p = agg_k(hs.reshape(NC * NP, H), sd3)
    return tc_last(p, degt, b3.reshape(1, D))


# speedup vs baseline: 1.5287x; 1.0498x over previous
import functools

import jax
import jax.numpy as jnp
from jax import lax
from jax.experimental import pallas as pl
from jax.experimental.pallas import tpu as pltpu
from jax.experimental.pallas import tpu_sc as plsc

NC = 2
NS = 16
NW = NC * NS
CHUNK = 128
K = 4
J = 2 * K
BLK = 1024


def _ceil_to(x, m):
    return -(-x // m) * m


def _deg_body(nch, dst2_hbm, zeros1_hbm, deg_hbm, di_v, ones_v, acc):
    c = lax.axis_index("c")
    s = lax.axis_index("s")
    rows = zeros1_hbm.shape[0] // NS
    rs = pl.ds(s * rows, rows)
    for k in range(CHUNK // 16):
        ones_v[pl.ds(k * 16, 16)] = jnp.ones((16,), jnp.float32)
    pltpu.sync_copy(zeros1_hbm.at[rs], acc.at[rs])
    plsc.subcore_barrier()
    rowbase = (c * NS + s) * nch
    pltpu.sync_copy(dst2_hbm.at[pl.ds(rowbase, nch)], di_v)

    def it(i, carry):
        pltpu.sync_copy(ones_v, acc.at[di_v.at[i]], add=True)
        return carry

    lax.fori_loop(0, nch, it, 0)
    plsc.subcore_barrier()
    pltpu.sync_copy(acc.at[rs], deg_hbm.at[c, rs])


def _agg_body(nch, npad, hs_hbm, sd3_hbm, p_hbm, sd_v, acc, *bufs):
    rows_b = bufs[:K]
    isem = bufs[K:K + J]
    gsem = bufs[K + J:K + J + K]
    ssem = bufs[K + J + K:]
    c = lax.axis_index("c")
    s = lax.axis_index("s")
    rows = npad // NS
    rs = pl.ds(s * rows, rows)

    pltpu.sync_copy(hs_hbm.at[pl.ds(c * npad + s * rows, rows)], acc.at[rs])
    plsc.subcore_barrier()
    rowbase = s * nch

    for v in range(J):
        pltpu.async_copy(sd3_hbm.at[c, rowbase + v], sd_v.at[v], isem[v])
    for v in range(K):
        pltpu.make_async_copy(sd3_hbm.at[c, rowbase], sd_v.at[v],
                              isem[v]).wait()
        pltpu.async_copy(hs_hbm.at[sd_v.at[v, 0]], rows_b[v], gsem[v])

    def grp(g, carry):
        for v in range(J):
            i = g * J + v
            k = v % K
            j2 = (v + K) % J
            pltpu.make_async_copy(hs_hbm.at[sd_v.at[v, 0]], rows_b[k],
                                  gsem[k]).wait()
            pltpu.async_copy(rows_b[k], acc.at[sd_v.at[v, 1]], ssem[k],
                             add=True)
            pltpu.make_async_copy(rows_b[k], acc.at[sd_v.at[v, 1]],
                                  ssem[k]).wait()
            rl = rowbase + jnp.minimum(i + J, nch - 1)
            pltpu.async_copy(sd3_hbm.at[c, rl], sd_v.at[v], isem[v])
            pltpu.make_async_copy(sd3_hbm.at[c, rowbase], sd_v.at[j2],
                                  isem[j2]).wait()
            pltpu.async_copy(hs_hbm.at[sd_v.at[j2, 0]], rows_b[k], gsem[k])
        return carry

    lax.fori_loop(0, nch // J, grp, 0)
    for k in range(K):
        pltpu.make_async_copy(hs_hbm.at[sd_v.at[0, 0]], rows_b[k],
                              gsem[k]).wait()
    for t in range(K, J):
        pltpu.make_async_copy(sd3_hbm.at[c, rowbase], sd_v.at[t],
                              isem[t]).wait()
    plsc.subcore_barrier()
    pltpu.sync_copy(acc.at[rs], p_hbm.at[c, rs])


def _dinv(degt_blk):
    return lax.rsqrt(1.0 + degt_blk[:, 0:1] + degt_blk[:, 1:2])


def _split_store(o_ref, val, h):
    o_ref[0] = val[:, :h]
    o_ref[1] = val[:, h:]


def _tc_first_body(x_ref, w_ref, degt_ref, o_ref):
    dinv = _dinv(degt_ref[...])
    hs = dinv * jnp.dot(x_ref[...], w_ref[...],
                        preferred_element_type=jnp.float32)
    _split_store(o_ref, hs, w_ref.shape[1] // 2)


def _tc_mid_body(p_ref, degt_ref, b_ref, w_ref, o_ref):
    dinv = _dinv(degt_ref[...])
    agg = jnp.concatenate([p_ref[0], p_ref[1]], axis=1)
    y = jnp.maximum(dinv * agg + b_ref[...], 0.0)
    hs = dinv * jnp.dot(y, w_ref[...], preferred_element_type=jnp.float32)
    _split_store(o_ref, hs, w_ref.shape[1] // 2)


def _tc_last_body(p_ref, degt_ref, b_ref, o_ref):
    dinv = _dinv(degt_ref[...])
    agg = jnp.concatenate([p_ref[0], p_ref[1]], axis=1)
    o_ref[...] = dinv * agg + b_ref[...]


def kernel(x, edge_index, W1, b1, W2, b2, W3, b3):
    N, D = x.shape
    H = D // 2
    E = edge_index.shape[1]
    NP = _ceil_to(N, NS * 8 * 2)
    EP = _ceil_to(E, NS * CHUNK * J)
    nch = EP // (NS * CHUNK)

    x = x.astype(jnp.float32)
    src = edge_index[0].astype(jnp.int32)
    dst = edge_index[1].astype(jnp.int32)
    pad = EP - E
    if pad:
        cyc = jnp.arange(pad, dtype=jnp.int32) % jnp.int32(NP - N)
        src = jnp.concatenate([src, cyc % jnp.int32(N)])
        dst = jnp.concatenate([dst, jnp.int32(N) + cyc])
    src2 = src.reshape(EP // CHUNK, CHUNK)
    dst2 = dst.reshape(EP // CHUNK, CHUNK)
    sd3 = jnp.stack([jnp.stack([src2, dst2], 1),
                     jnp.stack([src2 + jnp.int32(NP), dst2], 1)])
    zeros1 = jnp.zeros((NP,), jnp.float32)

    mesh = plsc.VectorSubcoreMesh(core_axis_name="c", subcore_axis_name="s")

    deg_k = pl.kernel(
        functools.partial(_deg_body, EP // (NW * CHUNK)),
        out_type=jax.ShapeDtypeStruct((NC, NP), jnp.float32),
        mesh=mesh,
        compiler_params=pltpu.CompilerParams(use_tc_tiling_on_sc=False),
        scratch_types=[
            pltpu.VMEM((EP // (NW * CHUNK), CHUNK), jnp.int32),
            pltpu.VMEM((CHUNK,), jnp.float32),
            pltpu.VMEM_SHARED((NP,), jnp.float32),
        ],
    )
    agg_k = pl.kernel(
        functools.partial(_agg_body, nch, NP),
        out_type=jax.ShapeDtypeStruct((NC, NP, H), jnp.float32),
        mesh=mesh,
        compiler_params=pltpu.CompilerParams(use_tc_tiling_on_sc=False),
        scratch_types=[
            pltpu.VMEM((J, 2, CHUNK), jnp.int32),
            pltpu.VMEM_SHARED((NP, H), jnp.float32),
        ] + [pltpu.VMEM((CHUNK, H), jnp.float32)] * K
          + [pltpu.SemaphoreType.DMA] * (J + 2 * K),
    )

    grid = (NP // BLK,) if NP % BLK == 0 else (-(-NP // BLK),)
    row_spec = pl.BlockSpec((BLK, D), lambda i: (i, 0))
    hs_spec = pl.BlockSpec((NC, BLK, H), lambda i: (0, i, 0))
    degt_spec = pl.BlockSpec((BLK, NC), lambda i: (i, 0))
    b_spec = pl.BlockSpec((1, D), lambda i: (0, 0))
    w_spec = pl.BlockSpec((D, D), lambda i: (0, 0))

    tc_first = pl.pallas_call(
        _tc_first_body, grid=grid,
        in_specs=[row_spec, w_spec, degt_spec], out_specs=hs_spec,
        out_shape=jax.ShapeDtypeStruct((NC, NP, H), jnp.float32))
    tc_mid = pl.pallas_call(
        _tc_mid_body, grid=grid,
        in_specs=[hs_spec, degt_spec, b_spec, w_spec], out_specs=hs_spec,
        out_shape=jax.ShapeDtypeStruct((NC, NP, H), jnp.float32))
    tc_last = pl.pallas_call(
        _tc_last_body, grid=grid,
        in_specs=[hs_spec, degt_spec, b_spec], out_specs=row_spec,
        out_shape=jax.ShapeDtypeStruct((N, D), jnp.float32))

    deg = deg_k(dst2, zeros1)
    degt = deg.T
    hs = tc_first(x, W1.astype(jnp.float32), degt)
    p = agg_k(hs.reshape(NC * NP, H), sd3)
    hs = tc_mid(p, degt, b1.reshape(1, D), W2.astype(jnp.float32))
    p = agg_k(hs.reshape(NC * NP, H), sd3)
    hs = tc_mid(p, degt, b2.reshape(1, D), W3.astype(jnp.float32))
    p = agg_k(hs.reshape(NC * NP, H), sd3)
    return tc_last(p, degt, b3.reshape(1, D))


# speedup vs baseline: 1.6472x; 1.0775x over previous
import functools

import jax
import jax.numpy as jnp
from jax import lax
from jax.experimental import pallas as pl
from jax.experimental.pallas import tpu as pltpu
from jax.experimental.pallas import tpu_sc as plsc

NC = 2
NS = 16
NW = NC * NS
CHUNK = 128
K = 4
J = 2 * K
BLK = 1024


def _ceil_to(x, m):
    return -(-x // m) * m


def _deg_body(nch, dst2_hbm, zeros1_hbm, deg_hbm, di_v, ones_v, acc):
    c = lax.axis_index("c")
    s = lax.axis_index("s")
    rows = zeros1_hbm.shape[0] // NS
    rs = pl.ds(s * rows, rows)
    for k in range(CHUNK // 16):
        ones_v[pl.ds(k * 16, 16)] = jnp.ones((16,), jnp.float32)
    pltpu.sync_copy(zeros1_hbm.at[rs], acc.at[rs])
    plsc.subcore_barrier()
    rowbase = (c * NS + s) * nch
    pltpu.sync_copy(dst2_hbm.at[pl.ds(rowbase, nch)], di_v)

    def it(i, carry):
        pltpu.sync_copy(ones_v, acc.at[di_v.at[i]], add=True)
        return carry

    lax.fori_loop(0, nch, it, 0)
    plsc.subcore_barrier()
    pltpu.sync_copy(acc.at[rs], deg_hbm.at[c, rs])


def _agg_body(nch, npad, h, hs_hbm, sd3_hbm, p_hbm, sd_v, acc, *bufs):
    rows_b = bufs[:K]
    isem = bufs[K:K + J]
    gsem = bufs[K + J:K + J + K]
    ssem = bufs[K + J + K:]
    c = lax.axis_index("c")
    s = lax.axis_index("s")
    rows = npad // NS
    rs = pl.ds(s * rows, rows)
    col = pl.ds(c * h, h)

    pltpu.sync_copy(hs_hbm.at[pl.ds(c * npad + s * rows, rows)], acc.at[rs])
    plsc.subcore_barrier()
    rowbase = s * nch

    for v in range(J):
        pltpu.async_copy(sd3_hbm.at[c, rowbase + v], sd_v.at[v], isem[v])
    for v in range(K):
        pltpu.make_async_copy(sd3_hbm.at[c, rowbase], sd_v.at[v],
                              isem[v]).wait()
        pltpu.async_copy(hs_hbm.at[sd_v.at[v, 0]], rows_b[v], gsem[v])

    def grp(g, carry):
        for v in range(J):
            i = g * J + v
            k = v % K
            j2 = (v + K) % J
            pltpu.make_async_copy(hs_hbm.at[sd_v.at[v, 0]], rows_b[k],
                                  gsem[k]).wait()
            pltpu.async_copy(rows_b[k], acc.at[sd_v.at[v, 1]], ssem[k],
                             add=True)
            pltpu.make_async_copy(rows_b[k], acc.at[sd_v.at[v, 1]],
                                  ssem[k]).wait()
            rl = rowbase + jnp.minimum(i + J, nch - 1)
            pltpu.async_copy(sd3_hbm.at[c, rl], sd_v.at[v], isem[v])
            pltpu.make_async_copy(sd3_hbm.at[c, rowbase], sd_v.at[j2],
                                  isem[j2]).wait()
            pltpu.async_copy(hs_hbm.at[sd_v.at[j2, 0]], rows_b[k],
                             gsem[k])
        return carry

    lax.fori_loop(0, nch // J, grp, 0)
    for k in range(K):
        pltpu.make_async_copy(hs_hbm.at[sd_v.at[0, 0]], rows_b[k],
                              gsem[k]).wait()
    for t in range(K, J):
        pltpu.make_async_copy(sd3_hbm.at[c, rowbase], sd_v.at[t],
                              isem[t]).wait()
    plsc.subcore_barrier()
    pltpu.sync_copy(acc.at[rs], p_hbm.at[rs, col])


def _dinv(degt_blk):
    return lax.rsqrt(1.0 + degt_blk[:, 0:1] + degt_blk[:, 1:2])


def _split_store(o_ref, val, h):
    o_ref[0] = val[:, :h]
    o_ref[1] = val[:, h:]


def _tc_first_body(x_ref, w_ref, degt_ref, o_ref):
    dinv = _dinv(degt_ref[...])
    hs = dinv * jnp.dot(x_ref[...], w_ref[...],
                        preferred_element_type=jnp.float32)
    _split_store(o_ref, hs, w_ref.shape[1] // 2)


def _tc_mid_body(p_ref, degt_ref, b_ref, w_ref, o_ref):
    dinv = _dinv(degt_ref[...])
    y = jnp.maximum(dinv * p_ref[...] + b_ref[...], 0.0)
    hs = dinv * jnp.dot(y, w_ref[...], preferred_element_type=jnp.float32)
    _split_store(o_ref, hs, w_ref.shape[1] // 2)


def _tc_last_body(p_ref, degt_ref, b_ref, o_ref):
    dinv = _dinv(degt_ref[...])
    o_ref[...] = dinv * p_ref[...] + b_ref[...]


def kernel(x, edge_index, W1, b1, W2, b2, W3, b3):
    N, D = x.shape
    H = D // 2
    E = edge_index.shape[1]
    NP = _ceil_to(N, NS * 8 * 2)
    EP = _ceil_to(E, NS * CHUNK * J)
    nch = EP // (NS * CHUNK)

    x = x.astype(jnp.float32)
    src = edge_index[0].astype(jnp.int32)
    dst = edge_index[1].astype(jnp.int32)
    pad = EP - E
    if pad:
        cyc = jnp.arange(pad, dtype=jnp.int32) % jnp.int32(NP - N)
        src = jnp.concatenate([src, cyc % jnp.int32(N)])
        dst = jnp.concatenate([dst, jnp.int32(N) + cyc])
    src2 = src.reshape(EP // CHUNK, CHUNK)
    dst2 = dst.reshape(EP // CHUNK, CHUNK)
    sd3 = jnp.stack([jnp.stack([src2, dst2], 1),
                     jnp.stack([src2 + jnp.int32(NP), dst2], 1)])
    zeros1 = jnp.zeros((NP,), jnp.float32)

    mesh = plsc.VectorSubcoreMesh(core_axis_name="c", subcore_axis_name="s")

    deg_k = pl.kernel(
        functools.partial(_deg_body, EP // (NW * CHUNK)),
        out_type=jax.ShapeDtypeStruct((NC, NP), jnp.float32),
        mesh=mesh,
        compiler_params=pltpu.CompilerParams(use_tc_tiling_on_sc=False),
        scratch_types=[
            pltpu.VMEM((EP // (NW * CHUNK), CHUNK), jnp.int32),
            pltpu.VMEM((CHUNK,), jnp.float32),
            pltpu.VMEM_SHARED((NP,), jnp.float32),
        ],
    )
    agg_k = pl.kernel(
        functools.partial(_agg_body, nch, NP, H),
        out_type=jax.ShapeDtypeStruct((NP, D), jnp.float32),
        mesh=mesh,
        compiler_params=pltpu.CompilerParams(use_tc_tiling_on_sc=False),
        scratch_types=[
            pltpu.VMEM((J, 2, CHUNK), jnp.int32),
            pltpu.VMEM_SHARED((NP, H), jnp.float32),
        ] + [pltpu.VMEM((CHUNK, H), jnp.float32)] * K
          + [pltpu.SemaphoreType.DMA] * (J + 2 * K),
    )

    grid = (NP // BLK,) if NP % BLK == 0 else (-(-NP // BLK),)
    row_spec = pl.BlockSpec((BLK, D), lambda i: (i, 0))
    hs_spec = pl.BlockSpec((NC, BLK, H), lambda i: (0, i, 0))
    degt_spec = pl.BlockSpec((BLK, NC), lambda i: (i, 0))
    b_spec = pl.BlockSpec((1, D), lambda i: (0, 0))
    w_spec = pl.BlockSpec((D, D), lambda i: (0, 0))

    tc_first = pl.pallas_call(
        _tc_first_body, grid=grid,
        in_specs=[row_spec, w_spec, degt_spec], out_specs=hs_spec,
        out_shape=jax.ShapeDtypeStruct((NC, NP, H), jnp.float32))
    tc_mid = pl.pallas_call(
        _tc_mid_body, grid=grid,
        in_specs=[row_spec, degt_spec, b_spec, w_spec], out_specs=hs_spec,
        out_shape=jax.ShapeDtypeStruct((NC, NP, H), jnp.float32))
    tc_last = pl.pallas_call(
        _tc_last_body, grid=grid,
        in_specs=[row_spec, degt_spec, b_spec], out_specs=row_spec,
        out_shape=jax.ShapeDtypeStruct((N, D), jnp.float32))

    deg = deg_k(dst2, zeros1)
    degt = deg.T
    hs = tc_first(x, W1.astype(jnp.float32), degt)
    p = agg_k(hs.reshape(NC * NP, H), sd3)
    hs = tc_mid(p, degt, b1.reshape(1, D), W2.astype(jnp.float32))
    p = agg_k(hs.reshape(NC * NP, H), sd3)
    hs = tc_mid(p, degt, b2.reshape(1, D), W3.astype(jnp.float32))
    p = agg_k(hs.reshape(NC * NP, H), sd3)
    return tc_last(p, degt, b3.reshape(1, D))
